# baseline (speedup 1.0000x reference)
import functools

import jax
import jax.numpy as jnp
from jax import lax
from jax.experimental import pallas as pl
from jax.experimental.pallas import tpu as pltpu
from jax.experimental.pallas import tpu_sc as plsc

BATCH = 16384
FACTORS = 64
FPAD = 128
NUM_CORES = 2
NUM_SUBCORES = 16
NUM_WORKERS = NUM_CORES * NUM_SUBCORES
ROWS_PER_WORKER = BATCH // NUM_WORKERS
CHUNKS = 8
CHUNK_ROWS = ROWS_PER_WORKER // CHUNKS
NBUF = 2
TYPE_ROWS = 24
MOVIE_ROWS = 1302


def _gmf_body(user_hbm, movie_hbm, type_hbm, eu_hbm, em_hbm, et_hbm, w_hbm,
              b_hbm, out_hbm, uidx_v, midx_v, tidx_v, eu_v, em_v, etw_v,
              w_v, b_v, out_v, sem0, sem1, sem2, sem3):
    sems = [sem0, sem1]
    msems = [sem2, sem3]
    wid = lax.axis_index("s") * NUM_CORES + lax.axis_index("c")
    base = wid * ROWS_PER_WORKER

    for j in range(CHUNKS):
        off = base + j * CHUNK_ROWS
        pltpu.sync_copy(user_hbm.at[pl.ds(off, CHUNK_ROWS)], uidx_v.at[j])
        pltpu.sync_copy(movie_hbm.at[pl.ds(off, CHUNK_ROWS)], midx_v.at[j])
    pltpu.sync_copy(type_hbm.at[pl.ds(base, ROWS_PER_WORKER)], tidx_v)

    copies = [None] * CHUNKS
    mcopies = [None] * CHUNKS
    for j in range(NBUF):
        copies[j] = pltpu.async_copy(eu_hbm.at[uidx_v.at[j]], eu_v.at[j],
                                     sems[j])
        mcopies[j] = pltpu.async_copy(em_hbm.at[midx_v.at[j]], em_v.at[j],
                                      msems[j])
    pltpu.sync_copy(et_hbm, etw_v)
    pltpu.sync_copy(w_hbm, w_v)
    pltpu.sync_copy(b_hbm, b_v)

    for t in range(TYPE_ROWS):
        for k in range(FACTORS // 16):
            sl = pl.ds(t * FACTORS + k * 16, 16)
            etw_v[sl] = etw_v[sl] * w_v[pl.ds(k * 16, 16)]

    lanes = lax.iota(jnp.int32, 16)
    ones = jnp.full((16,), 1, jnp.int32)
    cmask = jnp.full((16,), FACTORS - 1, jnp.int32)
    acc0 = b_v[...]
    fzeros = jnp.zeros((16,), jnp.float32)

    for j in range(CHUNKS):
        copies[j].wait()
        mcopies[j].wait()
        buf = j % NBUF

        def group_body(h, carry, j=j, buf=buf):
            off = j * CHUNK_ROWS + h * 16
            rows = h * 16 + lanes
            trow = tidx_v[pl.ds(off, 16)] * FACTORS
            col = lanes
            accs = [acc0, fzeros, fzeros, fzeros]
            for f in range(FACTORS):
                a = plsc.load_gather(eu_v.at[buf], [rows, col])
                m = plsc.load_gather(em_v.at[buf], [rows, col])
                t = plsc.load_gather(etw_v, [trow + col])
                accs[f % 4] = accs[f % 4] + a * m * t
                if f != FACTORS - 1:
                    col = (col + ones) & cmask
            out_v[pl.ds(off, 16)] = (accs[0] + accs[1]) + (accs[2] + accs[3])
            return carry

        lax.fori_loop(0, CHUNK_ROWS // 16, group_body, 0)

        if j + NBUF < CHUNKS:
            copies[j + NBUF] = pltpu.async_copy(
                eu_hbm.at[uidx_v.at[j + NBUF]], eu_v.at[buf], sems[buf])
            mcopies[j + NBUF] = pltpu.async_copy(
                em_hbm.at[midx_v.at[j + NBUF]], em_v.at[buf], msems[buf])

    pltpu.sync_copy(out_v, out_hbm.at[pl.ds(base, ROWS_PER_WORKER)])


@jax.jit
def _gmf(user, movie, type_id, embed_user, embed_movie, embed_type, w_flat, b):
    mesh = plsc.VectorSubcoreMesh(core_axis_name="c", subcore_axis_name="s")
    run = functools.partial(
        pl.kernel,
        out_type=jax.ShapeDtypeStruct((BATCH,), jnp.float32),
        mesh=mesh,
        scratch_types=[
            pltpu.VMEM((CHUNKS, CHUNK_ROWS), jnp.int32),
            pltpu.VMEM((CHUNKS, CHUNK_ROWS), jnp.int32),
            pltpu.VMEM((ROWS_PER_WORKER,), jnp.int32),
            pltpu.VMEM((NBUF, CHUNK_ROWS, FPAD), jnp.float32),
            pltpu.VMEM((NBUF, CHUNK_ROWS, FPAD), jnp.float32),
            pltpu.VMEM((TYPE_ROWS * FACTORS,), jnp.float32),
            pltpu.VMEM((FACTORS,), jnp.float32),
            pltpu.VMEM((16,), jnp.float32),
            pltpu.VMEM((ROWS_PER_WORKER,), jnp.float32),
            pltpu.SemaphoreType.DMA,
            pltpu.SemaphoreType.DMA,
            pltpu.SemaphoreType.DMA,
            pltpu.SemaphoreType.DMA,
        ],
        compiler_params=pltpu.CompilerParams(
            needs_layout_passes=False, use_tc_tiling_on_sc=True),
    )(_gmf_body)
    return run(user, movie, type_id, embed_user, embed_movie, embed_type,
               w_flat, b)


def kernel(user, movie, type_id, embed_user, embed_movie, embed_type, W, b):
    user = user.astype(jnp.int32)
    movie = movie.astype(jnp.int32)
    type_id = type_id.astype(jnp.int32)
    w_flat = W.reshape(-1).astype(jnp.float32)
    b_vec = jnp.broadcast_to(b.astype(jnp.float32).reshape(-1)[:1], (16,))
    eu_pad = jnp.pad(embed_user.astype(jnp.float32), ((0, 0), (0, FPAD - FACTORS)))
    em_pad = jnp.pad(embed_movie.astype(jnp.float32), ((0, 0), (0, FPAD - FACTORS)))
    et_flat = embed_type.astype(jnp.float32).reshape(-1)
    out = _gmf(user, movie, type_id, eu_pad, em_pad, et_flat, w_flat, b_vec)
    return out.reshape(-1, 1)


# speedup vs baseline: 1.0443x; 1.0443x over previous
import functools

import jax
import jax.numpy as jnp
from jax import lax
from jax.experimental import pallas as pl
from jax.experimental.pallas import tpu as pltpu
from jax.experimental.pallas import tpu_sc as plsc

BATCH = 16384
FACTORS = 64
FPAD = 128
NUM_CORES = 2
NUM_SUBCORES = 16
NUM_WORKERS = NUM_CORES * NUM_SUBCORES
ROWS_PER_WORKER = BATCH // NUM_WORKERS
CHUNKS = 4
CHUNK_ROWS = ROWS_PER_WORKER // CHUNKS
NBUF = 2
TYPE_ROWS = 24
MOVIE_ROWS = 1302


def _gmf_body(user_hbm, movie_hbm, type_hbm, eu_hbm, em_hbm, et_hbm, w_hbm,
              b_hbm, out_hbm, uidx_v, midx_v, tidx_v, eu_v, em_v, etw_v,
              w_v, b_v, out_v, sem0, sem1, sem2, sem3):
    sems = [sem0, sem1]
    msems = [sem2, sem3]
    wid = lax.axis_index("s") * NUM_CORES + lax.axis_index("c")
    base = wid * ROWS_PER_WORKER

    for j in range(CHUNKS):
        off = base + j * CHUNK_ROWS
        pltpu.sync_copy(user_hbm.at[pl.ds(off, CHUNK_ROWS)], uidx_v.at[j])
        pltpu.sync_copy(movie_hbm.at[pl.ds(off, CHUNK_ROWS)], midx_v.at[j])
    pltpu.sync_copy(type_hbm.at[pl.ds(base, ROWS_PER_WORKER)], tidx_v)

    copies = [None] * CHUNKS
    mcopies = [None] * CHUNKS
    for j in range(NBUF):
        copies[j] = pltpu.async_copy(eu_hbm.at[uidx_v.at[j]], eu_v.at[j],
                                     sems[j])
        mcopies[j] = pltpu.async_copy(em_hbm.at[midx_v.at[j]], em_v.at[j],
                                      msems[j])
    pltpu.sync_copy(et_hbm, etw_v)
    pltpu.sync_copy(w_hbm, w_v)
    pltpu.sync_copy(b_hbm, b_v)

    for t in range(TYPE_ROWS):
        for k in range(FACTORS // 16):
            sl = pl.ds(t * FACTORS + k * 16, 16)
            etw_v[sl] = etw_v[sl] * w_v[pl.ds(k * 16, 16)]

    lanes = lax.iota(jnp.int32, 16)
    ones = jnp.full((16,), 1, jnp.int32)
    cmask = jnp.full((16,), FACTORS - 1, jnp.int32)
    acc0 = b_v[...]
    fzeros = jnp.zeros((16,), jnp.float32)

    for j in range(CHUNKS):
        copies[j].wait()
        mcopies[j].wait()
        buf = j % NBUF

        def group_body(h, carry, j=j, buf=buf):
            off = j * CHUNK_ROWS + h * 16
            rows = h * 16 + lanes
            trow = tidx_v[pl.ds(off, 16)] * FACTORS
            col = lanes
            accs = [acc0, fzeros, fzeros, fzeros]
            for f in range(FACTORS):
                a = plsc.load_gather(eu_v.at[buf], [rows, col])
                m = plsc.load_gather(em_v.at[buf], [rows, col])
                t = plsc.load_gather(etw_v, [trow + col])
                accs[f % 4] = accs[f % 4] + a * m * t
                if f != FACTORS - 1:
                    col = (col + ones) & cmask
            out_v[pl.ds(off, 16)] = (accs[0] + accs[1]) + (accs[2] + accs[3])
            return carry

        lax.fori_loop(0, CHUNK_ROWS // 16, group_body, 0)

        if j + NBUF < CHUNKS:
            copies[j + NBUF] = pltpu.async_copy(
                eu_hbm.at[uidx_v.at[j + NBUF]], eu_v.at[buf], sems[buf])
            mcopies[j + NBUF] = pltpu.async_copy(
                em_hbm.at[midx_v.at[j + NBUF]], em_v.at[buf], msems[buf])

    pltpu.sync_copy(out_v, out_hbm.at[pl.ds(base, ROWS_PER_WORKER)])


@jax.jit
def _gmf(user, movie, type_id, embed_user, embed_movie, embed_type, w_flat, b):
    mesh = plsc.VectorSubcoreMesh(core_axis_name="c", subcore_axis_name="s")
    run = functools.partial(
        pl.kernel,
        out_type=jax.ShapeDtypeStruct((BATCH,), jnp.float32),
        mesh=mesh,
        scratch_types=[
            pltpu.VMEM((CHUNKS, CHUNK_ROWS), jnp.int32),
            pltpu.VMEM((CHUNKS, CHUNK_ROWS), jnp.int32),
            pltpu.VMEM((ROWS_PER_WORKER,), jnp.int32),
            pltpu.VMEM((NBUF, CHUNK_ROWS, FPAD), jnp.float32),
            pltpu.VMEM((NBUF, CHUNK_ROWS, FPAD), jnp.float32),
            pltpu.VMEM((TYPE_ROWS * FACTORS,), jnp.float32),
            pltpu.VMEM((FACTORS,), jnp.float32),
            pltpu.VMEM((16,), jnp.float32),
            pltpu.VMEM((ROWS_PER_WORKER,), jnp.float32),
            pltpu.SemaphoreType.DMA,
            pltpu.SemaphoreType.DMA,
            pltpu.SemaphoreType.DMA,
            pltpu.SemaphoreType.DMA,
        ],
        compiler_params=pltpu.CompilerParams(
            needs_layout_passes=False, use_tc_tiling_on_sc=True),
    )(_gmf_body)
    return run(user, movie, type_id, embed_user, embed_movie, embed_type,
               w_flat, b)


def kernel(user, movie, type_id, embed_user, embed_movie, embed_type, W, b):
    user = user.astype(jnp.int32)
    movie = movie.astype(jnp.int32)
    type_id = type_id.astype(jnp.int32)
    w_flat = W.reshape(-1).astype(jnp.float32)
    b_vec = jnp.broadcast_to(b.astype(jnp.float32).reshape(-1)[:1], (16,))
    eu_pad = jnp.pad(embed_user.astype(jnp.float32), ((0, 0), (0, FPAD - FACTORS)))
    em_pad = jnp.pad(embed_movie.astype(jnp.float32), ((0, 0), (0, FPAD - FACTORS)))
    et_flat = embed_type.astype(jnp.float32).reshape(-1)
    out = _gmf(user, movie, type_id, eu_pad, em_pad, et_flat, w_flat, b_vec)
    return out.reshape(-1, 1)
